# baseline (speedup 1.0000x reference)
import functools

import jax
import jax.numpy as jnp
from jax import lax
from jax.experimental import pallas as pl
from jax.experimental.pallas import tpu as pltpu
from jax.experimental.pallas import tpu_sc as plsc

D = 64
NT = 15
B = 1024
P = 10000
P_PAD = 10240
NC, NS = 2, 16
NW = NC * NS
CB = B // NW
PB = P_PAD // NW
CHUNK = 80
NCH = PB // CHUNK
PN = 2048

NBIG_P = 3
NBIG_C = 2

SMALL_COLS = (1, 2, 3, 5, 6, 7, 8, 9, 10, 12, 13, 14)
SMALL_V = (5, 5, 113, 1002, 3, 5, 32, 13, 11, 33, 65, 257)
NSM = len(SMALL_COLS)


def _sc_gather_body(cidx, pidx, ptab, postal, pname,
                    cfeat, pfeat, cidx_v, pidx_v, prows_v, gsem, ssem):
    ptabs = [ptab, postal, pname]
    ctabs = [postal, pname]

    wid = lax.axis_index("s") * NC + lax.axis_index("c")

    pltpu.sync_copy(cidx.at[wid], cidx_v)
    pltpu.sync_copy(pidx.at[wid], pidx_v)

    cg = [
        pltpu.async_copy(ctabs[t].at[cidx_v.at[t]],
                         prows_v.at[0, t, pl.ds(0, CB)], gsem)
        for t in range(NBIG_C)
    ]
    for op in cg:
        op.wait()
    cs = [
        pltpu.async_copy(prows_v.at[0, t, pl.ds(0, CB)],
                         cfeat.at[t, pl.ds(wid * CB, CB)], ssem)
        for t in range(NBIG_C)
    ]
    for op in cs:
        op.wait()

    pg = []
    for ci in range(NCH):
        off = ci * CHUNK
        pg.append([
            pltpu.async_copy(ptabs[t].at[pidx_v.at[t, pl.ds(off, CHUNK)]],
                             prows_v.at[ci, t], gsem)
            for t in range(NBIG_P)
        ])
    sops = []
    for ci in range(NCH):
        off = ci * CHUNK
        for op in pg[ci]:
            op.wait()
        for t in range(NBIG_P):
            sops.append(pltpu.async_copy(
                prows_v.at[ci, t],
                pfeat.at[t, pl.ds(wid * PB + off, CHUNK)], ssem))
    for op in sops:
        op.wait()


_sc_gather = functools.partial(
    pl.kernel,
    out_type=(jax.ShapeDtypeStruct((NBIG_C, B, 2 * D), jnp.float32),
              jax.ShapeDtypeStruct((NBIG_P, P_PAD, 2 * D), jnp.float32)),
    mesh=plsc.VectorSubcoreMesh(core_axis_name="c", subcore_axis_name="s",
                                num_cores=NC, num_subcores=NS),
    scratch_types=[
        pltpu.VMEM((NBIG_C, CB), jnp.int32),
        pltpu.VMEM((NBIG_P, PB), jnp.int32),
        pltpu.VMEM((NCH, NBIG_P, CHUNK, 2 * D), jnp.float32),
        pltpu.SemaphoreType.DMA,
        pltpu.SemaphoreType.DMA,
    ],
    compiler_params=pltpu.CompilerParams(use_tc_tiling_on_sc=False),
)(_sc_gather_body)


VB = 8192


def _tr_body(x_ref, o_ref):
    y = jnp.transpose(x_ref[...], (1, 0))
    o_ref[...] = jnp.concatenate([y, y], axis=1)


def _tab_transpose(tabT):
    v = tabT.shape[1]
    vpad = -(-v // VB) * VB
    return pl.pallas_call(
        _tr_body,
        grid=(vpad // VB,),
        in_specs=[pl.BlockSpec((64, VB), lambda j: (0, j))],
        out_specs=pl.BlockSpec((VB, 2 * D), lambda j: (j, 0)),
        out_shape=jax.ShapeDtypeStruct((vpad, 2 * D), jnp.float32),
    )(tabT)


def _cid_body(cid_smem, cid_v, tabT, out_ref, blk, sem):
    def fire(i, carry):
        for u in range(4):
            k = cid_smem[4 * i + u]
            tc = pl.multiple_of((k // 128) * 128, 128)
            pltpu.make_async_copy(tabT.at[:, pl.ds(tc, 128)],
                                  blk.at[4 * i + u], sem).start()
        return carry

    lax.fori_loop(0, B // 4, fire, 0)
    pltpu.make_async_copy(blk, blk, sem).wait()

    lane = cid_v[...] % 128
    iota = lax.broadcasted_iota(jnp.int32, (1, 128), 1)
    for c0 in range(0, B, 128):
        oh = (lane[c0:c0 + 128][:, None] == iota).astype(jnp.float32)
        out_ref[c0:c0 + 128, :] = jnp.sum(
            blk[c0:c0 + 128] * oh[:, None, :], axis=2)


def _cid_gather(cid, tabT):
    return pl.pallas_call(
        _cid_body,
        in_specs=[
            pl.BlockSpec(memory_space=pltpu.SMEM),
            pl.BlockSpec(memory_space=pltpu.VMEM),
            pl.BlockSpec(memory_space=pl.ANY),
        ],
        out_specs=pl.BlockSpec(memory_space=pltpu.VMEM),
        out_shape=jax.ShapeDtypeStruct((B, D), jnp.float32),
        scratch_shapes=[
            pltpu.VMEM((B, D, 128), jnp.float32),
            pltpu.SemaphoreType.DMA,
        ],
        compiler_params=pltpu.CompilerParams(
            vmem_limit_bytes=100 * 1024 * 1024),
    )(cid, cid, tabT)


def _mm_body(c0_ref, cbig_ref, pbig_ref, cidx_ref, pidx_ref, *rest):
    small_refs = rest[:NSM]
    o_ref = rest[NSM]
    cfull = rest[NSM + 1]
    pfull = rest[NSM + 2]
    j = pl.program_id(0)

    @pl.when(j == 0)
    def _():
        cfull[:, 0:D] = c0_ref[...].astype(jnp.bfloat16)
        cfull[:, D:2 * D] = cbig_ref[0][:, :D].astype(jnp.bfloat16)
        cfull[:, 2 * D:3 * D] = cbig_ref[1][:, :D].astype(jnp.bfloat16)
        for t in range(NSM):
            ohc = (cidx_ref[t][:, None] ==
                   lax.broadcasted_iota(jnp.int32, (B, SMALL_V[t]), 1))
            cfull[:, (3 + t) * D:(4 + t) * D] = lax.dot_general(
                ohc.astype(jnp.bfloat16),
                small_refs[t][...].astype(jnp.bfloat16),
                dimension_numbers=(((1,), (0,)), ((), ())),
                preferred_element_type=jnp.float32,
            ).astype(jnp.bfloat16)

    pfull[:, 0:D] = pbig_ref[0][:, :D].astype(jnp.bfloat16)
    pfull[:, D:2 * D] = pbig_ref[1][:, :D].astype(jnp.bfloat16)
    pfull[:, 2 * D:3 * D] = pbig_ref[2][:, :D].astype(jnp.bfloat16)
    for t in range(NSM):
        ohp = (pidx_ref[t][:, None] ==
               lax.broadcasted_iota(jnp.int32, (PN, SMALL_V[t]), 1))
        pfull[:, (3 + t) * D:(4 + t) * D] = lax.dot_general(
            ohp.astype(jnp.bfloat16), small_refs[t][...].astype(jnp.bfloat16),
            dimension_numbers=(((1,), (0,)), ((), ())),
            preferred_element_type=jnp.float32,
        ).astype(jnp.bfloat16)

    o_ref[...] = lax.dot_general(
        pfull[...], cfull[...],
        dimension_numbers=(((1,), (1,)), ((), ())),
        preferred_element_type=jnp.float32)


def _matmul(cfeat0, cfeat_big, pfeat_big, cidx_s, pidx_s, smalls):
    return pl.pallas_call(
        _mm_body,
        grid=(P_PAD // PN,),
        in_specs=[
            pl.BlockSpec((B, D), lambda j: (0, 0)),
            pl.BlockSpec((NBIG_C, B, 2 * D), lambda j: (0, 0, 0)),
            pl.BlockSpec((NBIG_P, PN, 2 * D), lambda j: (0, j, 0)),
            pl.BlockSpec((NSM, B), lambda j: (0, 0)),
            pl.BlockSpec((NSM, PN), lambda j: (0, j)),
        ] + [pl.BlockSpec((v, D), lambda j: (0, 0)) for v in SMALL_V],
        out_specs=pl.BlockSpec((PN, B), lambda j: (j, 0)),
        out_shape=jax.ShapeDtypeStruct((P, B), jnp.float32),
        scratch_shapes=[pltpu.VMEM((B, NT * D), jnp.bfloat16),
                        pltpu.VMEM((PN, NT * D), jnp.bfloat16)],
        compiler_params=pltpu.CompilerParams(
            vmem_limit_bytes=100 * 1024 * 1024),
    )(cfeat0, cfeat_big, pfeat_big, cidx_s, pidx_s, *smalls)


def kernel(Customer_data, Product_data, customer_table, product_table,
           price_table, age_table, colour_table, department_table,
           prod_name_table, sales_channel_table, season_table, day_table,
           month_table, year_table, club_table, fashion_table, postal_table,
           graphical_table):
    cdat = Customer_data.astype(jnp.int32)
    pdat = jnp.pad(Product_data.astype(jnp.int32), ((0, P_PAD - P), (0, 0)))

    cidx = cdat[:, (4, 11)].T.reshape(NBIG_C, NW, CB).transpose(1, 0, 2)
    pidx = pdat[:, (0, 4, 11)].T.reshape(NBIG_P, NW, PB).transpose(1, 0, 2)

    cfeat_big, pfeat_big = _sc_gather(
        cidx, pidx,
        _tab_transpose(product_table.T),
        _tab_transpose(postal_table.T),
        _tab_transpose(prod_name_table.T))
    cfeat0 = _cid_gather(cdat[:, 0], customer_table.T)

    smalls = (club_table, fashion_table, age_table, price_table,
              sales_channel_table, season_table, day_table, month_table,
              year_table, graphical_table, colour_table, department_table)
    cidx_s = cdat[:, SMALL_COLS].T
    pidx_s = pdat[:, SMALL_COLS].T
    return _matmul(cfeat0, cfeat_big, pfeat_big, cidx_s, pidx_s, smalls).T


# speedup vs baseline: 1.0273x; 1.0273x over previous
import functools

import jax
import jax.numpy as jnp
from jax import lax
from jax.experimental import pallas as pl
from jax.experimental.pallas import tpu as pltpu
from jax.experimental.pallas import tpu_sc as plsc

D = 64
NT = 15
B = 1024
P = 10000
P_PAD = 10240
NC, NS = 2, 16
NW = NC * NS
CB = B // NW
PB = P_PAD // NW
CHUNK = 80
NCH = PB // CHUNK
PN = 2048

NBIG_P = 3
NBIG_C = 2

SMALL_COLS = (1, 2, 3, 5, 6, 7, 8, 9, 10, 12, 13, 14)
SMALL_V = (5, 5, 113, 1002, 3, 5, 32, 13, 11, 33, 65, 257)
NSM = len(SMALL_COLS)


def _sc_gather_body(cidx, pidx, ptab, postal, pname,
                    cfeat, pfeat, cidx_v, pidx_v, prows_v, gsem, ssem):
    ptabs = [ptab, postal, pname]
    ctabs = [postal, pname]

    wid = lax.axis_index("s") * NC + lax.axis_index("c")

    pltpu.sync_copy(cidx.at[wid], cidx_v)
    pltpu.sync_copy(pidx.at[wid], pidx_v)

    cg = [
        pltpu.async_copy(ctabs[t].at[cidx_v.at[t]],
                         prows_v.at[0, t, pl.ds(0, CB)], gsem)
        for t in range(NBIG_C)
    ]
    for op in cg:
        op.wait()
    cs = [
        pltpu.async_copy(prows_v.at[0, t, pl.ds(0, CB)],
                         cfeat.at[t, pl.ds(wid * CB, CB)], ssem)
        for t in range(NBIG_C)
    ]
    for op in cs:
        op.wait()

    pg = []
    for ci in range(NCH):
        off = ci * CHUNK
        pg.append([
            pltpu.async_copy(ptabs[t].at[pidx_v.at[t, pl.ds(off, CHUNK)]],
                             prows_v.at[ci, t], gsem)
            for t in range(NBIG_P)
        ])
    sops = []
    for ci in range(NCH):
        off = ci * CHUNK
        for op in pg[ci]:
            op.wait()
        for t in range(NBIG_P):
            sops.append(pltpu.async_copy(
                prows_v.at[ci, t],
                pfeat.at[t, pl.ds(wid * PB + off, CHUNK)], ssem))
    for op in sops:
        op.wait()


_sc_gather = functools.partial(
    pl.kernel,
    out_type=(jax.ShapeDtypeStruct((NBIG_C, B, 2 * D), jnp.float32),
              jax.ShapeDtypeStruct((NBIG_P, P_PAD, 2 * D), jnp.float32)),
    mesh=plsc.VectorSubcoreMesh(core_axis_name="c", subcore_axis_name="s",
                                num_cores=NC, num_subcores=NS),
    scratch_types=[
        pltpu.VMEM((NBIG_C, CB), jnp.int32),
        pltpu.VMEM((NBIG_P, PB), jnp.int32),
        pltpu.VMEM((NCH, NBIG_P, CHUNK, 2 * D), jnp.float32),
        pltpu.SemaphoreType.DMA,
        pltpu.SemaphoreType.DMA,
    ],
    compiler_params=pltpu.CompilerParams(use_tc_tiling_on_sc=False),
)(_sc_gather_body)


VB = 8192


def _tr_body(x_ref, o_ref):
    y = jnp.transpose(x_ref[...], (1, 0))
    o_ref[...] = jnp.concatenate([y[:VB // 2], y[VB // 2:]], axis=1)


def _tab_transpose(tabT):
    v = tabT.shape[1]
    vpad = -(-v // VB) * VB
    return pl.pallas_call(
        _tr_body,
        grid=(vpad // VB,),
        in_specs=[pl.BlockSpec((64, VB), lambda j: (0, j))],
        out_specs=pl.BlockSpec((VB // 2, 2 * D), lambda j: (j, 0)),
        out_shape=jax.ShapeDtypeStruct((vpad // 2, 2 * D), jnp.float32),
    )(tabT)


def _cid_body(cid_smem, cid_v, tabT, out_ref, blk, sem):
    def fire(i, carry):
        for u in range(4):
            k = cid_smem[4 * i + u]
            tc = pl.multiple_of((k // 128) * 128, 128)
            pltpu.make_async_copy(tabT.at[:, pl.ds(tc, 128)],
                                  blk.at[4 * i + u], sem).start()
        return carry

    lax.fori_loop(0, B // 4, fire, 0)
    pltpu.make_async_copy(blk, blk, sem).wait()

    lane = cid_v[...] % 128
    iota = lax.broadcasted_iota(jnp.int32, (1, 128), 1)
    for c0 in range(0, B, 128):
        oh = (lane[c0:c0 + 128][:, None] == iota).astype(jnp.float32)
        out_ref[c0:c0 + 128, :] = jnp.sum(
            blk[c0:c0 + 128] * oh[:, None, :], axis=2)


def _cid_gather(cid, tabT):
    return pl.pallas_call(
        _cid_body,
        in_specs=[
            pl.BlockSpec(memory_space=pltpu.SMEM),
            pl.BlockSpec(memory_space=pltpu.VMEM),
            pl.BlockSpec(memory_space=pl.ANY),
        ],
        out_specs=pl.BlockSpec(memory_space=pltpu.VMEM),
        out_shape=jax.ShapeDtypeStruct((B, D), jnp.float32),
        scratch_shapes=[
            pltpu.VMEM((B, D, 128), jnp.float32),
            pltpu.SemaphoreType.DMA,
        ],
        compiler_params=pltpu.CompilerParams(
            vmem_limit_bytes=100 * 1024 * 1024),
    )(cid, cid, tabT)


def _mm_body(c0_ref, cbig_ref, pbig_ref, chalf_ref, phalf_ref,
             cidx_ref, pidx_ref, *rest):
    small_refs = rest[:NSM]
    o_ref = rest[NSM]
    cfull = rest[NSM + 1]
    pfull = rest[NSM + 2]
    j = pl.program_id(0)

    def half_sel(rows, half):
        n = rows.shape[0]
        ones = lax.broadcasted_iota(jnp.int32, (n, D), 1) * 0 + 1
        hmask = half[:, None] == ones
        return jnp.where(hmask, rows[:, D:2 * D], rows[:, 0:D])

    @pl.when(j == 0)
    def _():
        cfull[:, 0:D] = c0_ref[...].astype(jnp.bfloat16)
        cfull[:, D:2 * D] = half_sel(cbig_ref[0], chalf_ref[0]).astype(jnp.bfloat16)
        cfull[:, 2 * D:3 * D] = half_sel(cbig_ref[1], chalf_ref[1]).astype(jnp.bfloat16)
        for t in range(NSM):
            ohc = (cidx_ref[t][:, None] ==
                   lax.broadcasted_iota(jnp.int32, (B, SMALL_V[t]), 1))
            cfull[:, (3 + t) * D:(4 + t) * D] = lax.dot_general(
                ohc.astype(jnp.bfloat16),
                small_refs[t][...].astype(jnp.bfloat16),
                dimension_numbers=(((1,), (0,)), ((), ())),
                preferred_element_type=jnp.float32,
            ).astype(jnp.bfloat16)

    pfull[:, 0:D] = half_sel(pbig_ref[0], phalf_ref[0]).astype(jnp.bfloat16)
    pfull[:, D:2 * D] = half_sel(pbig_ref[1], phalf_ref[1]).astype(jnp.bfloat16)
    pfull[:, 2 * D:3 * D] = half_sel(pbig_ref[2], phalf_ref[2]).astype(jnp.bfloat16)
    for t in range(NSM):
        ohp = (pidx_ref[t][:, None] ==
               lax.broadcasted_iota(jnp.int32, (PN, SMALL_V[t]), 1))
        pfull[:, (3 + t) * D:(4 + t) * D] = lax.dot_general(
            ohp.astype(jnp.bfloat16), small_refs[t][...].astype(jnp.bfloat16),
            dimension_numbers=(((1,), (0,)), ((), ())),
            preferred_element_type=jnp.float32,
        ).astype(jnp.bfloat16)

    o_ref[...] = lax.dot_general(
        pfull[...], cfull[...],
        dimension_numbers=(((1,), (1,)), ((), ())),
        preferred_element_type=jnp.float32)


def _matmul(cfeat0, cfeat_big, pfeat_big, chalf, phalf, cidx_s, pidx_s, smalls):
    return pl.pallas_call(
        _mm_body,
        grid=(P_PAD // PN,),
        in_specs=[
            pl.BlockSpec((B, D), lambda j: (0, 0)),
            pl.BlockSpec((NBIG_C, B, 2 * D), lambda j: (0, 0, 0)),
            pl.BlockSpec((NBIG_P, PN, 2 * D), lambda j: (0, j, 0)),
            pl.BlockSpec((NBIG_C, B), lambda j: (0, 0)),
            pl.BlockSpec((NBIG_P, PN), lambda j: (0, j)),
            pl.BlockSpec((NSM, B), lambda j: (0, 0)),
            pl.BlockSpec((NSM, PN), lambda j: (0, j)),
        ] + [pl.BlockSpec((v, D), lambda j: (0, 0)) for v in SMALL_V],
        out_specs=pl.BlockSpec((PN, B), lambda j: (j, 0)),
        out_shape=jax.ShapeDtypeStruct((P, B), jnp.float32),
        scratch_shapes=[pltpu.VMEM((B, NT * D), jnp.bfloat16),
                        pltpu.VMEM((PN, NT * D), jnp.bfloat16)],
        compiler_params=pltpu.CompilerParams(
            vmem_limit_bytes=100 * 1024 * 1024),
    )(cfeat0, cfeat_big, pfeat_big, chalf, phalf, cidx_s, pidx_s, *smalls)


def kernel(Customer_data, Product_data, customer_table, product_table,
           price_table, age_table, colour_table, department_table,
           prod_name_table, sales_channel_table, season_table, day_table,
           month_table, year_table, club_table, fashion_table, postal_table,
           graphical_table):
    cdat = Customer_data.astype(jnp.int32)
    pdat = jnp.pad(Product_data.astype(jnp.int32), ((0, P_PAD - P), (0, 0)))

    craw = cdat[:, (4, 11)].T
    praw = pdat[:, (0, 4, 11)].T
    crow = (craw // VB) * (VB // 2) + craw % (VB // 2)
    prow = (praw // VB) * (VB // 2) + praw % (VB // 2)
    chalf = (craw % VB) // (VB // 2)
    phalf = (praw % VB) // (VB // 2)
    cidx = crow.reshape(NBIG_C, NW, CB).transpose(1, 0, 2)
    pidx = prow.reshape(NBIG_P, NW, PB).transpose(1, 0, 2)

    cfeat_big, pfeat_big = _sc_gather(
        cidx, pidx,
        _tab_transpose(product_table.T),
        _tab_transpose(postal_table.T),
        _tab_transpose(prod_name_table.T))
    cfeat0 = _cid_gather(cdat[:, 0], customer_table.T)

    smalls = (club_table, fashion_table, age_table, price_table,
              sales_channel_table, season_table, day_table, month_table,
              year_table, graphical_table, colour_table, department_table)
    cidx_s = cdat[:, SMALL_COLS].T
    pidx_s = pdat[:, SMALL_COLS].T
    return _matmul(cfeat0, cfeat_big, pfeat_big, chalf, phalf,
                   cidx_s, pidx_s, smalls).T
